# baseline (speedup 1.0000x reference)
import functools

import jax
import jax.numpy as jnp
from jax import lax
from jax.experimental import pallas as pl
from jax.experimental.pallas import tpu as pltpu
from jax.experimental.pallas import tpu_sc as plsc

D = 64
HC = 2
HW = D // HC
BPU = 2
KPAD, DPAD = 56, 128


def _build(B, K, NC, NS):
    NW = NC * NS
    ROWS = BPU * K
    assert ROWS <= 128
    U = B // BPU // NW
    assert U * BPU * NW == B and U % 2 == 0

    mesh = plsc.VectorSubcoreMesh(core_axis_name="c", subcore_axis_name="s")

    gbufs = [pltpu.VMEM((ROWS, HW), jnp.float32) for _ in range(2 * HC)]

    @functools.partial(
        pl.kernel,
        out_type=jax.ShapeDtypeStruct((B, KPAD, DPAD), jnp.float32),
        mesh=mesh,
        compiler_params=pltpu.CompilerParams(use_tc_tiling_on_sc=False),
        scratch_types=[pltpu.VMEM((U, ROWS), jnp.int32)] + gbufs
        + [pltpu.SemaphoreType.DMA] * 4,
    )
    def gather_kernel(t0_hbm, t1_hbm, idx_hbm, out_hbm, idx_v, *scr):
        gbuf = (scr[0:HC], scr[HC:2 * HC])
        gsem = (scr[2 * HC], scr[2 * HC + 1])
        osem = (scr[2 * HC + 2], scr[2 * HC + 3])
        thbm = (t0_hbm, t1_hbm)

        wid = lax.axis_index("s") * NC + lax.axis_index("c")
        b0 = wid * U * BPU

        pltpu.sync_copy(idx_hbm.at[pl.ds(wid * U, U)], idx_v)

        def fire_gather(b, u):
            for c in range(HC):
                pltpu.make_async_copy(
                    thbm[c].at[idx_v.at[u]], gbuf[b][c], gsem[b]).start()

        def wait_gather(b):
            for c in range(HC):
                pltpu.make_async_copy(
                    thbm[c].at[idx_v.at[0]], gbuf[b][c], gsem[b]).wait()

        def fire_flush(b, u):
            for j in range(BPU):
                for c in range(HC):
                    pltpu.make_async_copy(
                        gbuf[b][c].at[pl.ds(j * K, K)],
                        out_hbm.at[b0 + u * BPU + j, pl.ds(0, K),
                                   pl.ds(c * HW, HW)],
                        osem[b]).start()

        def wait_flush(b):
            for j in range(BPU):
                for c in range(HC):
                    pltpu.make_async_copy(
                        gbuf[b][c].at[pl.ds(0, K)],
                        out_hbm.at[0, pl.ds(0, K), pl.ds(0, HW)],
                        osem[b]).wait()

        fire_gather(0, 0)

        def body(i, carry):
            for b in (0, 1):
                u = 2 * i + b

                @pl.when(u + 1 < U)
                def _():
                    fire_gather(1 - b, u + 1)

                wait_gather(b)

                @pl.when(u >= 2)
                def _():
                    wait_flush(b)

                fire_flush(b, u)
            return carry

        lax.fori_loop(0, U // 2, body, 0)
        wait_flush(0)
        wait_flush(1)

    return gather_kernel


def kernel(input, indices):
    B, K = indices.shape
    info = plsc.get_sparse_core_info()
    NC, NS = info.num_cores, info.num_subcores

    idx2d = indices.astype(jnp.int32).reshape(B // BPU, BPU * K)
    halves = [input[:, c * HW:(c + 1) * HW] for c in range(HC)]
    out = _build(B, K, NC, NS)(*halves, idx2d)
    return out[:, :K, :D]


# speedup vs baseline: 1.8278x; 1.8278x over previous
import functools

import jax
import jax.numpy as jnp
from jax import lax
from jax.experimental import pallas as pl
from jax.experimental.pallas import tpu as pltpu
from jax.experimental.pallas import tpu_sc as plsc

D = 64
BPU = 2
KPAD, DPAD = 56, 128


def _build(B, K, NC, NS):
    NW = NC * NS
    ROWS = BPU * K
    assert ROWS <= 128
    U = B // BPU // NW
    assert U * BPU * NW == B and U % 2 == 0

    mesh = plsc.VectorSubcoreMesh(core_axis_name="c", subcore_axis_name="s")

    @functools.partial(
        pl.kernel,
        out_type=jax.ShapeDtypeStruct((B, KPAD, DPAD), jnp.float32),
        mesh=mesh,
        compiler_params=pltpu.CompilerParams(use_tc_tiling_on_sc=False),
        scratch_types=[
            pltpu.VMEM((U, ROWS), jnp.int32),
            pltpu.VMEM((ROWS, D), jnp.float32),
            pltpu.VMEM((ROWS, D), jnp.float32),
            pltpu.SemaphoreType.DMA,
            pltpu.SemaphoreType.DMA,
            pltpu.SemaphoreType.DMA,
            pltpu.SemaphoreType.DMA,
        ],
    )
    def gather_kernel(table_hbm, idx_hbm, out_hbm, idx_v, g0, g1,
                      gs0, gs1, os0, os1):
        gbuf = (g0, g1)
        gsem = (gs0, gs1)
        osem = (os0, os1)

        wid = lax.axis_index("s") * NC + lax.axis_index("c")
        b0 = wid * U * BPU

        pltpu.sync_copy(idx_hbm.at[pl.ds(wid * U, U)], idx_v)

        def fire_gather(b, u):
            pltpu.make_async_copy(
                table_hbm.at[idx_v.at[u]], gbuf[b], gsem[b]).start()

        def wait_gather(b):
            pltpu.make_async_copy(
                table_hbm.at[idx_v.at[0]], gbuf[b], gsem[b]).wait()

        def fire_flush(b, u):
            for j in range(BPU):
                pltpu.make_async_copy(
                    gbuf[b].at[pl.ds(j * K, K)],
                    out_hbm.at[b0 + u * BPU + j, pl.ds(0, K), pl.ds(0, D)],
                    osem[b]).start()

        def wait_flush(b):
            for j in range(BPU):
                pltpu.make_async_copy(
                    gbuf[b].at[pl.ds(0, K)],
                    out_hbm.at[0, pl.ds(0, K), pl.ds(0, D)],
                    osem[b]).wait()

        fire_gather(0, 0)

        def body(i, carry):
            for b in (0, 1):
                u = 2 * i + b

                @pl.when(u + 1 < U)
                def _():
                    fire_gather(1 - b, u + 1)

                wait_gather(b)

                @pl.when(u >= 2)
                def _():
                    wait_flush(b)

                fire_flush(b, u)
            return carry

        lax.fori_loop(0, U // 2, body, 0)
        wait_flush(0)
        wait_flush(1)

    return gather_kernel


def kernel(input, indices):
    B, K = indices.shape
    info = plsc.get_sparse_core_info()
    NC, NS = info.num_cores, info.num_subcores

    idx2d = indices.astype(jnp.int32).reshape(B // BPU, BPU * K)
    out = _build(B, K, NC, NS)(input, idx2d)
    return out[:, :K, :D]
